# baseline (device time: 167259 ns/iter reference)
import jax
import jax.numpy as jnp
from jax import lax
from jax.experimental import pallas as pl
from jax.experimental.pallas import tpu as pltpu

N_DEV = 4
NSUB = 4
R, L = 0, 1


def kernel(A, B):
    m, _ = A.shape
    _, n = B.shape
    mc = m // N_DEV
    nq = n // (2 * NSUB)

    def body(a_ref, b_ref, out_ref, sbuf_r, sbuf_l, rbuf_r, rbuf_l,
             rs_ss_r, rs_rs_r, rs_ss_l, rs_rs_l,
             ag_ss_r, ag_rs_r, ag_ss_l, ag_rs_l):
        my = lax.axis_index("i")
        left = (my - 1) % N_DEV
        right = (my + 1) % N_DEV

        barrier = pltpu.get_barrier_semaphore()
        for nbr in (left, right):
            pl.semaphore_signal(barrier, inc=1, device_id=(nbr,),
                                device_id_type=pl.DeviceIdType.MESH)
        pl.semaphore_wait(barrier, 2)

        def partial(c, q):
            return jnp.dot(a_ref[pl.ds(c * mc, mc), :],
                           b_ref[:, q * nq:(q + 1) * nq],
                           preferred_element_type=jnp.float32)

        def rs_rdma(d, b, s):
            sbuf, rbuf = (sbuf_r, rbuf_r) if d == R else (sbuf_l, rbuf_l)
            ss, rs = (rs_ss_r, rs_rs_r) if d == R else (rs_ss_l, rs_rs_l)
            return pltpu.make_async_remote_copy(
                src_ref=sbuf.at[b],
                dst_ref=rbuf.at[b, s % 2],
                send_sem=ss.at[b, s],
                recv_sem=rs.at[b, s],
                device_id=(right if d == R else left,),
                device_id_type=pl.DeviceIdType.MESH,
            )

        def ag_rdma(d, b, h):
            stripe = (my - h) % N_DEV if d == R else (my + h) % N_DEV
            q = b if d == R else NSUB + b
            region = out_ref.at[pl.ds(stripe * mc, mc), pl.ds(q * nq, nq)]
            ss, rs = (ag_ss_r, ag_rs_r) if d == R else (ag_ss_l, ag_rs_l)
            return pltpu.make_async_remote_copy(
                src_ref=region, dst_ref=region,
                send_sem=ss.at[b, h],
                recv_sem=rs.at[b, h],
                device_id=(right if d == R else left,),
                device_id_type=pl.DeviceIdType.MESH,
            )

        first = (my - 1) % N_DEV
        firstl = (my + 1) % N_DEV

        for b in range(NSUB):
            sbuf_r[b] = partial(first, b)
            rs_rdma(R, b, 0).start()
            sbuf_l[b] = partial(firstl, NSUB + b)
            rs_rdma(L, b, 0).start()

        for s in (1, 2):
            cr = (my - 1 - s) % N_DEV
            cl = (my + 1 + s) % N_DEV
            for b in range(NSUB):
                pr = partial(cr, b)
                rs_rdma(R, b, s - 1).wait_send()
                rs_rdma(R, b, s - 1).wait_recv()
                sbuf_r[b] = pr + rbuf_r[b, (s - 1) % 2]
                rs_rdma(R, b, s).start()

                pll = partial(cl, NSUB + b)
                rs_rdma(L, b, s - 1).wait_send()
                rs_rdma(L, b, s - 1).wait_recv()
                sbuf_l[b] = pll + rbuf_l[b, (s - 1) % 2]
                rs_rdma(L, b, s).start()

        rows = pl.ds(my * mc, mc)
        for b in range(NSUB):
            pr = partial(my, b)
            rs_rdma(R, b, 2).wait_recv()
            acc = rbuf_r[b, 0] + pr
            out_ref[rows, b * nq:(b + 1) * nq] = acc * (
                1.0 / (1.0 + jnp.exp(-acc)))
            ag_rdma(R, b, 0).start()

            pll = partial(my, NSUB + b)
            rs_rdma(L, b, 2).wait_recv()
            acc = rbuf_l[b, 0] + pll
            out_ref[rows, (NSUB + b) * nq:(NSUB + b + 1) * nq] = acc * (
                1.0 / (1.0 + jnp.exp(-acc)))
            ag_rdma(L, b, 0).start()

        for h in (1, 2):
            for b in range(NSUB):
                ag_rdma(R, b, h - 1).wait_recv()
                ag_rdma(R, b, h).start()
                ag_rdma(L, b, h - 1).wait_recv()
                ag_rdma(L, b, h).start()

        for b in range(NSUB):
            ag_rdma(R, b, 2).wait_recv()
            ag_rdma(L, b, 2).wait_recv()
            rs_rdma(R, b, 2).wait_send()
            rs_rdma(L, b, 2).wait_send()
            for h in (0, 1, 2):
                ag_rdma(R, b, h).wait_send()
                ag_rdma(L, b, h).wait_send()

    return pl.pallas_call(
        body,
        out_shape=jax.ShapeDtypeStruct((m, n), jnp.float32),
        in_specs=[
            pl.BlockSpec(memory_space=pltpu.VMEM),
            pl.BlockSpec(memory_space=pltpu.VMEM),
        ],
        out_specs=pl.BlockSpec(memory_space=pltpu.VMEM),
        scratch_shapes=[
            pltpu.VMEM((NSUB, mc, nq), jnp.float32),
            pltpu.VMEM((NSUB, mc, nq), jnp.float32),
            pltpu.VMEM((NSUB, 2, mc, nq), jnp.float32),
            pltpu.VMEM((NSUB, 2, mc, nq), jnp.float32),
            pltpu.SemaphoreType.DMA((NSUB, 3)),
            pltpu.SemaphoreType.DMA((NSUB, 3)),
            pltpu.SemaphoreType.DMA((NSUB, 3)),
            pltpu.SemaphoreType.DMA((NSUB, 3)),
            pltpu.SemaphoreType.DMA((NSUB, 3)),
            pltpu.SemaphoreType.DMA((NSUB, 3)),
            pltpu.SemaphoreType.DMA((NSUB, 3)),
            pltpu.SemaphoreType.DMA((NSUB, 3)),
        ],
        compiler_params=pltpu.CompilerParams(collective_id=0),
    )(A, B)


# device time: 99849 ns/iter; 1.6751x vs baseline; 1.6751x over previous
import jax
import jax.numpy as jnp
from jax import lax
from jax.experimental import pallas as pl
from jax.experimental.pallas import tpu as pltpu

N_DEV = 4
NSUB = 4
R, L = 0, 1


def kernel(A, B):
    m, k = A.shape
    _, n = B.shape
    mc = m // N_DEV
    nq = n // (2 * NSUB)

    def body(a_ref, b_ref, out_ref, agbuf_r, agbuf_l,
             sbuf_r, sbuf_l, rbuf_r, rbuf_l,
             rs_ss_r, rs_rs_r, rs_ss_l, rs_rs_l,
             ag_ss_r, ag_rs_r, ag_ss_l, ag_rs_l):
        my = lax.axis_index("i")
        left = (my - 1) % N_DEV
        right = (my + 1) % N_DEV

        barrier = pltpu.get_barrier_semaphore()
        for nbr in (left, right):
            pl.semaphore_signal(barrier, inc=1, device_id=(nbr,),
                                device_id_type=pl.DeviceIdType.MESH)
        pl.semaphore_wait(barrier, 2)

        def partial(c, q):
            return jnp.dot(a_ref[pl.ds(c * mc, mc), :],
                           b_ref[:, q * nq:(q + 1) * nq],
                           preferred_element_type=jnp.float32)

        def rs_rdma(d, b, s):
            sbuf, rbuf = (sbuf_r, rbuf_r) if d == R else (sbuf_l, rbuf_l)
            ss, rs = (rs_ss_r, rs_rs_r) if d == R else (rs_ss_l, rs_rs_l)
            return pltpu.make_async_remote_copy(
                src_ref=sbuf.at[b],
                dst_ref=rbuf.at[b, s % 2],
                send_sem=ss.at[b, s],
                recv_sem=rs.at[b, s],
                device_id=(right if d == R else left,),
                device_id_type=pl.DeviceIdType.MESH,
            )

        def ag_rdma(d, b, h):
            sbuf, agbuf = (sbuf_r, agbuf_r) if d == R else (sbuf_l, agbuf_l)
            ss, rs = (ag_ss_r, ag_rs_r) if d == R else (ag_ss_l, ag_rs_l)
            return pltpu.make_async_remote_copy(
                src_ref=sbuf.at[b] if h == 0 else agbuf.at[b, h - 1],
                dst_ref=agbuf.at[b, h],
                send_sem=ss.at[b, h],
                recv_sem=rs.at[b, h],
                device_id=(right if d == R else left,),
                device_id_type=pl.DeviceIdType.MESH,
            )

        def upcast(d, b, h):
            stripe = (my - 1 - h) % N_DEV if d == R else (my + 1 + h) % N_DEV
            q = b if d == R else NSUB + b
            agbuf = agbuf_r if d == R else agbuf_l
            out_ref[pl.ds(stripe * mc, mc), pl.ds(q * nq, nq)] = (
                agbuf[b, h].astype(jnp.float32))

        first = (my - 1) % N_DEV
        firstl = (my + 1) % N_DEV

        for b in range(NSUB):
            sbuf_r[b] = partial(first, b).astype(jnp.bfloat16)
            rs_rdma(R, b, 0).start()
            sbuf_l[b] = partial(firstl, NSUB + b).astype(jnp.bfloat16)
            rs_rdma(L, b, 0).start()

        for s in (1, 2):
            cr = (my - 1 - s) % N_DEV
            cl = (my + 1 + s) % N_DEV
            for b in range(NSUB):
                pr = partial(cr, b)
                rs_rdma(R, b, s - 1).wait_send()
                rs_rdma(R, b, s - 1).wait_recv()
                sbuf_r[b] = (
                    rbuf_r[b, (s - 1) % 2].astype(jnp.float32) + pr
                ).astype(jnp.bfloat16)
                rs_rdma(R, b, s).start()

                pll = partial(cl, NSUB + b)
                rs_rdma(L, b, s - 1).wait_send()
                rs_rdma(L, b, s - 1).wait_recv()
                sbuf_l[b] = (
                    rbuf_l[b, (s - 1) % 2].astype(jnp.float32) + pll
                ).astype(jnp.bfloat16)
                rs_rdma(L, b, s).start()

        rows = pl.ds(my * mc, mc)
        for b in range(NSUB):
            pr = partial(my, b)
            rs_rdma(R, b, 2).wait_recv()
            acc = rbuf_r[b, 0].astype(jnp.float32) + pr
            silu = acc * (1.0 / (1.0 + jnp.exp(-acc)))
            out_ref[rows, b * nq:(b + 1) * nq] = silu
            rs_rdma(R, b, 2).wait_send()
            sbuf_r[b] = silu.astype(jnp.bfloat16)
            ag_rdma(R, b, 0).start()

            pll = partial(my, NSUB + b)
            rs_rdma(L, b, 2).wait_recv()
            acc = rbuf_l[b, 0].astype(jnp.float32) + pll
            silu = acc * (1.0 / (1.0 + jnp.exp(-acc)))
            out_ref[rows, (NSUB + b) * nq:(NSUB + b + 1) * nq] = silu
            rs_rdma(L, b, 2).wait_send()
            sbuf_l[b] = silu.astype(jnp.bfloat16)
            ag_rdma(L, b, 0).start()

        for h in (1, 2):
            for b in range(NSUB):
                ag_rdma(R, b, h - 1).wait_recv()
                ag_rdma(R, b, h).start()
                upcast(R, b, h - 1)
                ag_rdma(L, b, h - 1).wait_recv()
                ag_rdma(L, b, h).start()
                upcast(L, b, h - 1)

        for b in range(NSUB):
            ag_rdma(R, b, 2).wait_recv()
            upcast(R, b, 2)
            ag_rdma(L, b, 2).wait_recv()
            upcast(L, b, 2)
            for h in (0, 1, 2):
                ag_rdma(R, b, h).wait_send()
                ag_rdma(L, b, h).wait_send()

    bf = jnp.bfloat16
    return pl.pallas_call(
        body,
        out_shape=jax.ShapeDtypeStruct((m, n), jnp.float32),
        in_specs=[
            pl.BlockSpec(memory_space=pltpu.VMEM),
            pl.BlockSpec(memory_space=pltpu.VMEM),
        ],
        out_specs=pl.BlockSpec(memory_space=pltpu.VMEM),
        scratch_shapes=[
            pltpu.VMEM((NSUB, 3, mc, nq), bf),
            pltpu.VMEM((NSUB, 3, mc, nq), bf),
            pltpu.VMEM((NSUB, mc, nq), bf),
            pltpu.VMEM((NSUB, mc, nq), bf),
            pltpu.VMEM((NSUB, 2, mc, nq), bf),
            pltpu.VMEM((NSUB, 2, mc, nq), bf),
            pltpu.SemaphoreType.DMA((NSUB, 3)),
            pltpu.SemaphoreType.DMA((NSUB, 3)),
            pltpu.SemaphoreType.DMA((NSUB, 3)),
            pltpu.SemaphoreType.DMA((NSUB, 3)),
            pltpu.SemaphoreType.DMA((NSUB, 3)),
            pltpu.SemaphoreType.DMA((NSUB, 3)),
            pltpu.SemaphoreType.DMA((NSUB, 3)),
            pltpu.SemaphoreType.DMA((NSUB, 3)),
        ],
        compiler_params=pltpu.CompilerParams(collective_id=0),
    )(A, B)


# device time: 94842 ns/iter; 1.7636x vs baseline; 1.0528x over previous
import jax
import jax.numpy as jnp
from jax import lax
from jax.experimental import pallas as pl
from jax.experimental.pallas import tpu as pltpu

N_DEV = 4
NSUB = 4
R, L = 0, 1


def kernel(A, B):
    m, k = A.shape
    _, n = B.shape
    mc = m // N_DEV
    nq = n // (2 * NSUB)
    bf = jnp.bfloat16

    def body(a_ref, b_ref, out_ref, a_bf, b_bf, agbuf_r, agbuf_l,
             sbuf_r, sbuf_l, rbuf_r, rbuf_l, stage, stage_sems,
             rs_ss_r, rs_rs_r, rs_ss_l, rs_rs_l,
             ag_ss_r, ag_rs_r, ag_ss_l, ag_rs_l):
        my = lax.axis_index("i")
        left = (my - 1) % N_DEV
        right = (my + 1) % N_DEV

        barrier = pltpu.get_barrier_semaphore()
        for nbr in (left, right):
            pl.semaphore_signal(barrier, inc=1, device_id=(nbr,),
                                device_id_type=pl.DeviceIdType.MESH)

        def _downcast(i, _):
            rows = pl.ds(i * (m // 8), m // 8)
            a_bf[rows, :] = a_ref[rows, :].astype(bf)
            krows = pl.ds(i * (k // 8), k // 8)
            b_bf[krows, :] = b_ref[krows, :].astype(bf)
            return 0

        lax.fori_loop(0, 8, _downcast, 0)
        pl.semaphore_wait(barrier, 2)

        def partial(c, q):
            return jnp.dot(a_bf[pl.ds(c * mc, mc), :],
                           b_bf[:, pl.ds(q * nq, nq)],
                           preferred_element_type=jnp.float32)

        def rs_rdma(d, b, s):
            sbuf, rbuf = (sbuf_r, rbuf_r) if d == R else (sbuf_l, rbuf_l)
            ss, rs = (rs_ss_r, rs_rs_r) if d == R else (rs_ss_l, rs_rs_l)
            return pltpu.make_async_remote_copy(
                src_ref=sbuf.at[b],
                dst_ref=rbuf.at[b, s % 2],
                send_sem=ss.at[b, s],
                recv_sem=rs.at[b, s],
                device_id=(right if d == R else left,),
                device_id_type=pl.DeviceIdType.MESH,
            )

        def ag_rdma(d, b, h):
            sbuf, agbuf = (sbuf_r, agbuf_r) if d == R else (sbuf_l, agbuf_l)
            ss, rs = (ag_ss_r, ag_rs_r) if d == R else (ag_ss_l, ag_rs_l)
            return pltpu.make_async_remote_copy(
                src_ref=sbuf.at[b] if h == 0 else agbuf.at[b, h - 1],
                dst_ref=agbuf.at[b, h],
                send_sem=ss.at[b, h],
                recv_sem=rs.at[b, h],
                device_id=(right if d == R else left,),
                device_id_type=pl.DeviceIdType.MESH,
            )

        issued = []

        def _out_copy(slot, rows, cols):
            return pltpu.make_async_copy(
                stage.at[slot], out_ref.at[rows, cols], stage_sems.at[slot])

        def write_out(value, rows, cols):
            slot = len(issued) % 4
            if len(issued) >= 4:
                _out_copy(*issued[len(issued) - 4]).wait()
            stage[slot] = value
            _out_copy(slot, rows, cols).start()
            issued.append((slot, rows, cols))

        def upcast(d, b, h):
            stripe = (my - 1 - h) % N_DEV if d == R else (my + 1 + h) % N_DEV
            q = b if d == R else NSUB + b
            agbuf = agbuf_r if d == R else agbuf_l
            write_out(agbuf[b, h].astype(jnp.float32),
                      pl.ds(stripe * mc, mc), pl.ds(q * nq, nq))

        first = (my - 1) % N_DEV
        firstl = (my + 1) % N_DEV

        for b in range(NSUB):
            sbuf_r[b] = partial(first, b).astype(bf)
            rs_rdma(R, b, 0).start()
            sbuf_l[b] = partial(firstl, NSUB + b).astype(bf)
            rs_rdma(L, b, 0).start()

        for s in (1, 2):
            cr = (my - 1 - s) % N_DEV
            cl = (my + 1 + s) % N_DEV
            for b in range(NSUB):
                pr = partial(cr, b)
                rs_rdma(R, b, s - 1).wait_send()
                rs_rdma(R, b, s - 1).wait_recv()
                sbuf_r[b] = (
                    rbuf_r[b, (s - 1) % 2].astype(jnp.float32) + pr
                ).astype(bf)
                rs_rdma(R, b, s).start()

                pll = partial(cl, NSUB + b)
                rs_rdma(L, b, s - 1).wait_send()
                rs_rdma(L, b, s - 1).wait_recv()
                sbuf_l[b] = (
                    rbuf_l[b, (s - 1) % 2].astype(jnp.float32) + pll
                ).astype(bf)
                rs_rdma(L, b, s).start()

        rows = pl.ds(my * mc, mc)
        for b in range(NSUB):
            pr = partial(my, b)
            rs_rdma(R, b, 2).wait_recv()
            acc = rbuf_r[b, 0].astype(jnp.float32) + pr
            silu = acc * (1.0 / (1.0 + jnp.exp(-acc)))
            rs_rdma(R, b, 2).wait_send()
            sbuf_r[b] = silu.astype(bf)
            ag_rdma(R, b, 0).start()
            write_out(silu, rows, pl.ds(b * nq, nq))

            pll = partial(my, NSUB + b)
            rs_rdma(L, b, 2).wait_recv()
            acc = rbuf_l[b, 0].astype(jnp.float32) + pll
            silu = acc * (1.0 / (1.0 + jnp.exp(-acc)))
            rs_rdma(L, b, 2).wait_send()
            sbuf_l[b] = silu.astype(bf)
            ag_rdma(L, b, 0).start()
            write_out(silu, rows, pl.ds((NSUB + b) * nq, nq))

        for h in (1, 2):
            for b in range(NSUB):
                ag_rdma(R, b, h - 1).wait_recv()
                ag_rdma(R, b, h).start()
                upcast(R, b, h - 1)
                ag_rdma(L, b, h - 1).wait_recv()
                ag_rdma(L, b, h).start()
                upcast(L, b, h - 1)

        for b in range(NSUB):
            ag_rdma(R, b, 2).wait_recv()
            upcast(R, b, 2)
            ag_rdma(L, b, 2).wait_recv()
            upcast(L, b, 2)
            for h in (0, 1, 2):
                ag_rdma(R, b, h).wait_send()
                ag_rdma(L, b, h).wait_send()

        for i in range(max(0, len(issued) - 4), len(issued)):
            _out_copy(*issued[i]).wait()

    return pl.pallas_call(
        body,
        out_shape=jax.ShapeDtypeStruct((m, n), jnp.float32),
        in_specs=[
            pl.BlockSpec(memory_space=pltpu.VMEM),
            pl.BlockSpec(memory_space=pltpu.VMEM),
        ],
        out_specs=pl.BlockSpec(memory_space=pl.ANY),
        scratch_shapes=[
            pltpu.VMEM((m, k), bf),
            pltpu.VMEM((k, n), bf),
            pltpu.VMEM((NSUB, 3, mc, nq), bf),
            pltpu.VMEM((NSUB, 3, mc, nq), bf),
            pltpu.VMEM((NSUB, mc, nq), bf),
            pltpu.VMEM((NSUB, mc, nq), bf),
            pltpu.VMEM((NSUB, 2, mc, nq), bf),
            pltpu.VMEM((NSUB, 2, mc, nq), bf),
            pltpu.VMEM((4, mc, nq), jnp.float32),
            pltpu.SemaphoreType.DMA((4,)),
            pltpu.SemaphoreType.DMA((NSUB, 3)),
            pltpu.SemaphoreType.DMA((NSUB, 3)),
            pltpu.SemaphoreType.DMA((NSUB, 3)),
            pltpu.SemaphoreType.DMA((NSUB, 3)),
            pltpu.SemaphoreType.DMA((NSUB, 3)),
            pltpu.SemaphoreType.DMA((NSUB, 3)),
            pltpu.SemaphoreType.DMA((NSUB, 3)),
            pltpu.SemaphoreType.DMA((NSUB, 3)),
        ],
        compiler_params=pltpu.CompilerParams(collective_id=0),
    )(A, B)
